# baseline (speedup 1.0000x reference)
import functools

import jax
import jax.numpy as jnp
from jax import lax
from jax.experimental import pallas as pl
from jax.experimental.pallas import tpu as pltpu
from jax.experimental.pallas import tpu_sc as plsc

B = 4
N = 262144
D = 32
G = 20
NV = G * G * G
HB = 8192
VOXEL = 0.05

NC = 2
NS = 16
PTS = N // NS
WA = 1024
NWA = PTS // WA
WB = 2048
NWB = N // WB
NTASK = 2 * D
KMAX = 4

_mesh = plsc.VectorSubcoreMesh(core_axis_name="c", subcore_axis_name="s")


@functools.partial(
    pl.kernel,
    mesh=_mesh,
    compiler_params=pltpu.CompilerParams(needs_layout_passes=False,
                                         use_tc_tiling_on_sc=False),
    out_type=[
        jax.ShapeDtypeStruct((B, D, NV), jnp.float32),
        jax.ShapeDtypeStruct((3, B, N), jnp.int32),
    ],
    scratch_types=[
        pltpu.VMEM((KMAX, HB), jnp.float32),
        pltpu.VMEM((2, 3, WA), jnp.float32),
        pltpu.VMEM((2, 3, WA), jnp.int32),
        pltpu.VMEM((2, WA), jnp.int32),
        pltpu.VMEM((2, 4, WB), jnp.float32),
        pltpu.VMEM((2, WB), jnp.int32),
        pltpu.VMEM((16, 512), jnp.float32),
        pltpu.VMEM((NS, 16), jnp.int32),
        pltpu.VMEM((16,), jnp.int32),
        pltpu.VMEM_SHARED((2, N), jnp.int32),
        pltpu.VMEM_SHARED((NS, 16, 512), jnp.float32),
        pltpu.VMEM_SHARED((2, 16, 512), jnp.float32),
        pltpu.VMEM_SHARED((NS, 16), jnp.int32),
        pltpu.SemaphoreType.DMA,
        pltpu.SemaphoreType.DMA,
        pltpu.SemaphoreType.DMA,
        pltpu.SemaphoreType.DMA,
        pltpu.SemaphoreType.DMA,
        pltpu.SemaphoreType.DMA,
        pltpu.SemaphoreType.DMA,
        pltpu.SemaphoreType.DMA,
    ],
)
def _voxel_kernel(xyz_hbm, feat_hbm, fout_hbm, vout_hbm,
                  hist, xyzw, va, la, fbw, lbw,
                  cntb, minall, minpub,
                  lin_sh, cnt_parts, cnt_sh, min_sh,
                  semf0, semf1, seml0, seml1,
                  semi0, semi1, semo0, semo1):
    c = lax.axis_index("c")
    s = lax.axis_index("s")
    iota = lax.iota(jnp.int32, 16)
    onesv = jnp.full((16,), 1.0, jnp.float32)
    zerov = jnp.zeros((16,), jnp.float32)
    bigf = jnp.full((16,), 1e30, jnp.float32)
    bigi = jnp.full((16,), 2**30, jnp.int32)
    vsize = jnp.float32(VOXEL)

    pt_base = s * PTS

    semis = (semi0, semi1)
    semos = (semo0, semo1)
    semfs = (semf0, semf1)
    semls = (seml0, seml1)

    def batch_body(bi, carry_b):
        b = c * 2 + bi

        def issue_xyz(w, p):
            base = pt_base + w * WA
            for d in range(3):
                pltpu.async_copy(xyz_hbm.at[d, b, pl.ds(base, WA)],
                                 xyzw.at[p, d], semis[p])

        def wait_xyz(p):
            for d in range(3):
                pltpu.make_async_copy(xyz_hbm.at[d, b, pl.ds(0, WA)],
                                      xyzw.at[p, d], semis[p]).wait()

        issue_xyz(0, 0)
        issue_xyz(1, 1)

        def a1_body(w2, carry):
            mx, my, mz = carry
            for p in range(2):
                w = 2 * w2 + p
                wait_xyz(p)
                for t in range(WA // 16):
                    sl = pl.ds(t * 16, 16)
                    mx = jnp.minimum(mx, xyzw[p, 0, sl])
                    my = jnp.minimum(my, xyzw[p, 1, sl])
                    mz = jnp.minimum(mz, xyzw[p, 2, sl])

                @pl.when(w + 2 < NWA)
                def _():
                    issue_xyz(w + 2, p)
            return mx, my, mz

        mx, my, mz = lax.fori_loop(0, NWA // 2, a1_body, (bigf, bigf, bigf))
        md = [jnp.min((m / vsize).astype(jnp.int32)) for m in (mx, my, mz)]

        pub = jnp.where(iota == 0, md[0],
                        jnp.where(iota == 1, md[1],
                                  jnp.where(iota == 2, md[2], bigi)))
        minpub[...] = pub
        pltpu.sync_copy(minpub, min_sh.at[s])
        plsc.subcore_barrier()
        pltpu.sync_copy(min_sh, minall)
        gmin = bigi
        for t2 in range(NS):
            gmin = jnp.minimum(gmin, minall[t2, :])
        gm0 = jnp.sum(jnp.where(iota == 0, gmin, 0))
        gm1 = jnp.sum(jnp.where(iota == 1, gmin, 0))
        gm2 = jnp.sum(jnp.where(iota == 2, gmin, 0))
        offset = gm0 * (G * G) + gm1 * G + gm2

        def czero_body(r2, carry):
            for j2 in range(32):
                cntb[r2, pl.ds(j2 * 16, 16)] = zerov
            return carry
        lax.fori_loop(0, 16, czero_body, 0)

        issue_xyz(0, 0)
        issue_xyz(1, 1)

        def a2_body(w2, carry):
            for p in range(2):
                w = 2 * w2 + p
                wait_xyz(p)
                for t in range(WA // 16):
                    sl = pl.ds(t * 16, 16)
                    cx = (xyzw[p, 0, sl] / vsize).astype(jnp.int32)
                    cy = (xyzw[p, 1, sl] / vsize).astype(jnp.int32)
                    cz = (xyzw[p, 2, sl] / vsize).astype(jnp.int32)
                    lin = cx * (G * G) + cy * G + cz - offset
                    lin = jnp.minimum(jnp.maximum(lin, 0), HB - 1)
                    la[p, sl] = lin
                    va[p, 0, sl] = cx - gm0
                    va[p, 1, sl] = cy - gm1
                    va[p, 2, sl] = cz - gm2
                    plsc.addupdate_scatter(
                        cntb, [lin >> 9, lin & 511], onesv)

                @pl.when(w + 2 < NWA)
                def _():
                    issue_xyz(w + 2, p)

                base = pt_base + w * WA
                pltpu.sync_copy(la.at[p], lin_sh.at[bi, pl.ds(base, WA)])
                for d in range(3):
                    pltpu.sync_copy(va.at[p, d],
                                    vout_hbm.at[d, b, pl.ds(base, WA)])
            return carry

        lax.fori_loop(0, NWA // 2, a2_body, 0)

        pltpu.sync_copy(cntb, cnt_parts.at[s])
        plsc.subcore_barrier()

        def red_body(t2, accs):
            pltpu.sync_copy(cnt_parts.at[t2, s], fbw.at[0, 0, pl.ds(0, 512)])
            return tuple(accs[i] + fbw[0, 0, pl.ds(i * 16, 16)] for i in range(32))

        accs = lax.fori_loop(0, NS, red_body, tuple([zerov] * 32))
        for i in range(32):
            cntb[0, pl.ds(i * 16, 16)] = accs[i]
        pltpu.sync_copy(cntb.at[0], cnt_sh.at[bi, s])
        plsc.subcore_barrier()
        return carry_b

    lax.fori_loop(0, 2, batch_body, 0)
    plsc.subcore_barrier()

    bi_b = lax.rem(s, 2)
    b_b = c * 2 + bi_b
    j0 = lax.div(s, 2)
    kvecs = [jnp.full((16,), k2, jnp.int32) for k2 in range(KMAX)]

    def issue_b(w, p):
        for k2 in range(KMAX):
            pltpu.async_copy(
                feat_hbm.at[b_b, j0 + 8 * k2, pl.ds(w * WB, WB)],
                fbw.at[p, k2], semfs[p])
        pltpu.async_copy(lin_sh.at[bi_b, pl.ds(w * WB, WB)],
                         lbw.at[p], semls[p])

    def zero_body(r, carry2):
        for k2 in range(KMAX):
            hist[k2, pl.ds(r * 16, 16)] = zerov
        return carry2
    lax.fori_loop(0, HB // 16, zero_body, 0)

    issue_b(0, 0)
    issue_b(1, 1)

    def win_body(w2, carry2):
        for p in range(2):
            w = 2 * w2 + p
            for k2 in range(KMAX):
                pltpu.make_async_copy(
                    feat_hbm.at[b_b, j0 + 8 * k2, pl.ds(0, WB)],
                    fbw.at[p, k2], semfs[p]).wait()
            pltpu.make_async_copy(lin_sh.at[bi_b, pl.ds(0, WB)],
                                  lbw.at[p], semls[p]).wait()
            for t in range(WB // 16):
                sl = pl.ds(t * 16, 16)
                idx = lbw[p, sl]
                for k2 in range(KMAX):
                    plsc.addupdate_scatter(hist, [kvecs[k2], idx],
                                           fbw[p, k2, sl])

            @pl.when(w + 2 < NWB)
            def _():
                issue_b(w + 2, p)
        return carry2

    lax.fori_loop(0, NWB // 2, win_body, 0)

    pltpu.sync_copy(cnt_sh.at[bi_b], cntb)

    def fin_task(k, carry):
        j = j0 + 8 * k

        def div_body(r, carry2):
            sl = pl.ds(r * 16, 16)
            row = r >> 5
            col = (lax.rem(r, 32)) * 16
            denom = jnp.maximum(cntb[row, pl.ds(col, 16)], 1.0)
            hist[k, sl] = hist[k, sl] / denom
            return carry2

        lax.fori_loop(0, NV // 16, div_body, 0)
        pltpu.sync_copy(hist.at[k, pl.ds(0, NV)],
                        fout_hbm.at[b_b, j, pl.ds(0, NV)])
        return carry

    lax.fori_loop(0, KMAX, fin_task, 0)
    plsc.subcore_barrier()


def kernel(xyz, features):
    xyz_t = jnp.transpose(xyz, (2, 0, 1))
    feat_t = jnp.transpose(features, (0, 2, 1))
    fout_t, vc_t = _voxel_kernel(xyz_t, feat_t)
    return (jnp.transpose(fout_t, (0, 2, 1)),
            jnp.transpose(vc_t, (1, 2, 0)))
